# Initial kernel scaffold; baseline (speedup 1.0000x reference)
import jax
import jax.numpy as jnp
from jax.experimental import pallas as pl


def kernel(x, edge_index, batch, W1, b1, W2, b2, W3, b3, Wf1, bf1, Wf2, bf2):
    raise NotImplementedError("write your pallas kernel here")


# speedup vs baseline: 5.8915x; 5.8915x over previous
import functools

import jax
import jax.numpy as jnp
from jax import lax
from jax.experimental import pallas as pl
from jax.experimental.pallas import tpu as pltpu
from jax.experimental.pallas import tpu_sc as plsc

N = 10000
NPAD = 10240
NT = 16
CHUNK = 128
NCH = 160
HCH = NCH // 2
E_PAD = NT * NCH * CHUNK
NROWS_T = NPAD // NT
DUMP = 10200
NB = NPAD // 128
NG = 64


def _sc_mesh():
    return plsc.VectorSubcoreMesh(core_axis_name="c", subcore_axis_name="s")


def _deg_call(dst3):
    @functools.partial(
        pl.kernel,
        mesh=_sc_mesh(),
        out_type=jax.ShapeDtypeStruct((2, NPAD, 16), jnp.float32),
        scratch_types=[
            pltpu.VMEM_SHARED((NPAD, 16), jnp.float32),
            pltpu.VMEM((HCH, CHUNK), jnp.int32),
            pltpu.VMEM((CHUNK, 16), jnp.float32),
            pltpu.VMEM((CHUNK, 16), jnp.float32),
        ],
    )
    def deg(dst_hbm, out_hbm, acc_sh, dst_v, ones_v, zero_v):
        c = lax.axis_index("c")
        s = lax.axis_index("s")
        r0 = s * NROWS_T
        sixteenth = jnp.full((16,), 1.0 / 16.0, jnp.float32)
        zeros16 = jnp.zeros((16,), jnp.float32)

        def fill(i, _):
            ones_v[i, :] = sixteenth
            zero_v[i, :] = zeros16
            return 0

        lax.fori_loop(0, CHUNK, fill, 0)

        def z(k, _):
            pltpu.sync_copy(zero_v, acc_sh.at[pl.ds(r0 + k * CHUNK, CHUNK)])
            return 0

        lax.fori_loop(0, NROWS_T // CHUNK, z, 0)
        pltpu.sync_copy(dst_hbm.at[s, pl.ds(c * HCH, HCH)], dst_v)
        plsc.subcore_barrier()

        def body(j, _):
            pltpu.sync_copy(ones_v, acc_sh.at[dst_v.at[j]], add=True)
            return 0

        lax.fori_loop(0, HCH, body, 0)
        plsc.subcore_barrier()
        pltpu.sync_copy(acc_sh.at[pl.ds(r0, NROWS_T)],
                        out_hbm.at[c, pl.ds(r0, NROWS_T)])

    return deg(dst3)


def _agg_call(hs, zeros_n, src3, dst3):
    @functools.partial(
        pl.kernel,
        mesh=_sc_mesh(),
        out_type=jax.ShapeDtypeStruct((2, NPAD, 128), jnp.float32),
        scratch_types=[
            pltpu.VMEM_SHARED((NPAD, 128), jnp.float32),
            pltpu.VMEM((HCH, CHUNK), jnp.int32),
            pltpu.VMEM((HCH, CHUNK), jnp.int32),
            pltpu.VMEM((CHUNK, 128), jnp.float32),
            pltpu.SemaphoreType.DMA,
        ],
    )
    def agg(hs_hbm, z_hbm, src_hbm, dst_hbm, out_hbm,
            acc_sh, src_v, dst_v, rows_v, sem):
        c = lax.axis_index("c")
        s = lax.axis_index("s")
        r0 = s * NROWS_T

        @pl.when(c == 0)
        def _():
            pltpu.sync_copy(hs_hbm.at[pl.ds(r0, NROWS_T)],
                            acc_sh.at[pl.ds(r0, NROWS_T)])

        @pl.when(c != 0)
        def _():
            pltpu.sync_copy(z_hbm.at[pl.ds(r0, NROWS_T)],
                            acc_sh.at[pl.ds(r0, NROWS_T)])

        pltpu.sync_copy(src_hbm.at[s, pl.ds(c * HCH, HCH)], src_v)
        pltpu.sync_copy(dst_hbm.at[s, pl.ds(c * HCH, HCH)], dst_v)
        plsc.subcore_barrier()

        def body(j, _):
            pltpu.async_copy(hs_hbm.at[src_v.at[j]], rows_v, sem).wait()
            pltpu.sync_copy(rows_v, acc_sh.at[dst_v.at[j]], add=True)
            return 0

        lax.fori_loop(0, HCH, body, 0)
        plsc.subcore_barrier()
        pltpu.sync_copy(acc_sh.at[pl.ds(r0, NROWS_T)],
                        out_hbm.at[c, pl.ds(r0, NROWS_T)])

    return agg(hs, zeros_n, src3, dst3)


def _k1(xp, W1, degp):
    def body(x_ref, w_ref, degp_ref, hs_ref, dinv_ref):
        i = pl.program_id(0)
        deg = (jnp.sum(degp_ref[0], axis=1, keepdims=True)
               + jnp.sum(degp_ref[1], axis=1, keepdims=True) + 1.0)
        dinv = lax.rsqrt(deg)
        row = lax.broadcasted_iota(jnp.int32, (128, 1), 0) + i * 128
        mask = row < N
        h = jnp.dot(x_ref[...], w_ref[...], preferred_element_type=jnp.float32)
        hs_ref[...] = jnp.where(mask, h * dinv, 0.0)
        dinv_ref[...] = jnp.broadcast_to(dinv, (128, 128))

    return pl.pallas_call(
        body,
        grid=(NB,),
        in_specs=[
            pl.BlockSpec((128, 128), lambda i: (i, 0)),
            pl.BlockSpec((128, 128), lambda i: (0, 0)),
            pl.BlockSpec((2, 128, 16), lambda i: (0, i, 0)),
        ],
        out_specs=[
            pl.BlockSpec((128, 128), lambda i: (i, 0)),
            pl.BlockSpec((128, 128), lambda i: (i, 0)),
        ],
        out_shape=[
            jax.ShapeDtypeStruct((NPAD, 128), jnp.float32),
            jax.ShapeDtypeStruct((NPAD, 128), jnp.float32),
        ],
    )(xp, W1, degp)


def _k2(accs, dinv, b, W, Din, Dout):
    na = len(accs)
    OH = Dout // 128

    def body(*refs):
        acc_refs = refs[:na]
        dinv_ref, b_ref, w_ref, out_ref = refs[na:]
        i = pl.program_id(0)
        d1 = dinv_ref[:, :1]
        parts = [a[0] + a[1] for a in acc_refs]
        accf = parts[0] if na == 1 else jnp.concatenate(parts, axis=1)
        row = lax.broadcasted_iota(jnp.int32, (128, 1), 0) + i * 128
        mask = row < N
        xv = jnp.where(mask, jnp.maximum(accf * d1 + b_ref[...], 0.0), 0.0)
        h = jnp.dot(xv, w_ref[...], preferred_element_type=jnp.float32)
        hs = jnp.where(mask, h * d1, 0.0)
        if OH == 1:
            out_ref[...] = hs
        else:
            out_ref[0] = hs[:, :128]
            out_ref[1] = hs[:, 128:]

    if OH == 1:
        out_spec = pl.BlockSpec((128, 128), lambda i: (i, 0))
        out_shape = jax.ShapeDtypeStruct((NPAD, 128), jnp.float32)
    else:
        out_spec = pl.BlockSpec((2, 128, 128), lambda i: (0, i, 0))
        out_shape = jax.ShapeDtypeStruct((2, NPAD, 128), jnp.float32)

    return pl.pallas_call(
        body,
        grid=(NB,),
        in_specs=(
            [pl.BlockSpec((2, 128, 128), lambda i: (0, i, 0))] * na
            + [
                pl.BlockSpec((128, 128), lambda i: (i, 0)),
                pl.BlockSpec((1, Din), lambda i: (0, 0)),
                pl.BlockSpec((Din, Dout), lambda i: (0, 0)),
            ]
        ),
        out_specs=out_spec,
        out_shape=out_shape,
    )(*accs, dinv, b, W)


def _k3(acc, dinv, b3, batch_p, Wf1, bf1, Wf2, bf2):
    def body(acc_ref, dinv_ref, b_ref, batch_ref, wf1_ref, bf1_ref,
             wf2_ref, bf2_ref, out_ref, sums, cnts):
        i = pl.program_id(0)

        @pl.when(i == 0)
        def _():
            sums[...] = jnp.zeros_like(sums)
            cnts[...] = jnp.zeros_like(cnts)

        d1 = dinv_ref[:, :1]
        accf = acc_ref[0] + acc_ref[1]
        row = lax.broadcasted_iota(jnp.int32, (128, 1), 0) + i * 128
        mask = row < N
        h = jnp.where(mask, jnp.maximum(accf * d1 + b_ref[...], 0.0), 0.0)
        g = lax.broadcasted_iota(jnp.int32, (NG, 128), 0)
        P = (g == batch_ref[0]).astype(jnp.float32)
        sums[...] = sums[...] + jnp.dot(P, h, preferred_element_type=jnp.float32)
        cnts[...] = cnts[...] + jnp.sum(P, axis=1, keepdims=True)

        @pl.when(i == NB - 1)
        def _():
            pooled = sums[...] / jnp.maximum(cnts[...], 1.0)
            g1 = jnp.maximum(
                jnp.dot(pooled, wf1_ref[...], preferred_element_type=jnp.float32)
                + bf1_ref[...], 0.0)
            g2 = (jnp.dot(g1, wf2_ref[...], preferred_element_type=jnp.float32)
                  + bf2_ref[...])
            m = jnp.max(g2, axis=1, keepdims=True)
            lse = m + jnp.log(jnp.sum(jnp.exp(g2 - m), axis=1, keepdims=True))
            out_ref[...] = g2 - lse

    return pl.pallas_call(
        body,
        grid=(NB,),
        in_specs=[
            pl.BlockSpec((2, 128, 128), lambda i: (0, i, 0)),
            pl.BlockSpec((128, 128), lambda i: (i, 0)),
            pl.BlockSpec((1, 128), lambda i: (0, 0)),
            pl.BlockSpec((1, 1, 128), lambda i: (i, 0, 0)),
            pl.BlockSpec((128, 128), lambda i: (0, 0)),
            pl.BlockSpec((1, 128), lambda i: (0, 0)),
            pl.BlockSpec((128, 10), lambda i: (0, 0)),
            pl.BlockSpec((1, 10), lambda i: (0, 0)),
        ],
        out_specs=pl.BlockSpec((NG, 10), lambda i: (0, 0)),
        out_shape=jax.ShapeDtypeStruct((NG, 10), jnp.float32),
        scratch_shapes=[
            pltpu.VMEM((NG, 128), jnp.float32),
            pltpu.VMEM((NG, 128), jnp.float32),
        ],
    )(acc, dinv, b3, batch_p, Wf1, bf1, Wf2, bf2)


def kernel(x, edge_index, batch, W1, b1, W2, b2, W3, b3, Wf1, bf1, Wf2, bf2):
    ei = edge_index.astype(jnp.int32)
    src = ei[0]
    dst = ei[1]
    padlen = E_PAD - src.shape[0]
    src_p = jnp.concatenate([src, jnp.full((padlen,), DUMP, jnp.int32)])
    dst_p = jnp.concatenate([dst, jnp.full((padlen,), DUMP, jnp.int32)])
    src3 = src_p.reshape(NT, NCH, CHUNK)
    dst3 = dst_p.reshape(NT, NCH, CHUNK)

    xp = jnp.pad(x, ((0, NPAD - N), (0, 0)))
    zeros_n = jnp.zeros((NPAD, 128), jnp.float32)
    batch_p = jnp.pad(batch.astype(jnp.int32), (0, NPAD - N),
                      constant_values=NG).reshape(NB, 1, 128)
    b1r = b1.reshape(1, -1)
    b2r = b2.reshape(1, -1)
    b3r = b3.reshape(1, -1)
    bf1r = bf1.reshape(1, -1)
    bf2r = bf2.reshape(1, -1)

    degp = _deg_call(dst3)
    hs1, dinv = _k1(xp, W1, degp)
    acc1 = _agg_call(hs1, zeros_n, src3, dst3)
    hs2 = _k2([acc1], dinv, b1r, W2, 128, 256)
    acc2a = _agg_call(hs2[0], zeros_n, src3, dst3)
    acc2b = _agg_call(hs2[1], zeros_n, src3, dst3)
    hs3 = _k2([acc2a, acc2b], dinv, b2r, W3, 256, 128)
    acc3 = _agg_call(hs3, zeros_n, src3, dst3)
    return _k3(acc3, dinv, b3r, batch_p, Wf1, bf1r, Wf2, bf2r)
